# Initial kernel scaffold; baseline (speedup 1.0000x reference)
import jax
import jax.numpy as jnp
from jax.experimental import pallas as pl


def kernel(neighbours, shape_features, W1s, W2s, bs, W1_out, W2_out, b_out):
    raise NotImplementedError("write your pallas kernel here")


# speedup vs baseline: 1.5287x; 1.5287x over previous
import functools

import jax
import jax.numpy as jnp
from jax import lax
from jax.experimental import pallas as pl
from jax.experimental.pallas import tpu as pltpu
from jax.experimental.pallas import tpu_sc as plsc

N, K, D = 10000, 32, 128
NC, NS, L = 2, 16, 16
NW = NC * NS
N_PAD = 10240
CPW = N_PAD // NW
ROWS_PER_STEP = 128
NODES_PER_STEP = ROWS_PER_STEP // K
STEPS = CPW * K // ROWS_PER_STEP
NJ = STEPS // 2


def _sc_body(nbr_ref, x_ref, out_ref, idx_v, rows_v, out_v, sem0, sem1):
    wid = lax.axis_index("s") * NC + lax.axis_index("c")
    base = wid * CPW

    pltpu.sync_copy(nbr_ref.at[pl.ds(wid * STEPS, STEPS)], idx_v)
    pltpu.async_copy(x_ref.at[idx_v.at[0]], rows_v.at[0], sem0)

    def compute_step(j, b, sem):
        pltpu.make_async_copy(x_ref.at[idx_v.at[j]], rows_v.at[b], sem).wait()
        rows = rows_v.at[b]

        def node_body(g, carry):
            node = j * NODES_PER_STEP + g
            r0 = g * K
            nd = D // L
            for d0 in range(0, nd, 4):
                ds_ = [pl.ds(d * L, L) for d in range(d0, d0 + 4)]
                accs = [rows[r0, s] for s in ds_]
                for k in range(1, K):
                    accs = [a + rows[r0 + k, s] for a, s in zip(accs, ds_)]
                for s, a in zip(ds_, accs):
                    out_v[node, s] = a
            return carry

        lax.fori_loop(0, NODES_PER_STEP, node_body, 0)

    def loop_body(jj, carry):
        j0 = jj * 2
        j1 = j0 + 1
        pltpu.async_copy(x_ref.at[idx_v.at[j1]], rows_v.at[1], sem1)
        compute_step(j0, 0, sem0)

        @pl.when(jj + 1 < NJ)
        def _():
            pltpu.async_copy(x_ref.at[idx_v.at[j0 + 2]], rows_v.at[0], sem0)

        compute_step(j1, 1, sem1)
        return carry

    lax.fori_loop(0, NJ, loop_body, 0)
    pltpu.sync_copy(out_v, out_ref.at[pl.ds(base, CPW)])


@functools.cache
def _gather_sum_kernel():
    return pl.kernel(
        _sc_body,
        out_type=jax.ShapeDtypeStruct((N_PAD, D), jnp.float32),
        mesh=plsc.VectorSubcoreMesh(core_axis_name="c", subcore_axis_name="s"),
        scratch_types=[
            pltpu.VMEM((STEPS, 128), jnp.int32),
            pltpu.VMEM((2, ROWS_PER_STEP, D), jnp.float32),
            pltpu.VMEM((CPW, D), jnp.float32),
            pltpu.SemaphoreType.DMA,
            pltpu.SemaphoreType.DMA,
        ],
    )


def _gather_sum(nbr, x):
    return _gather_sum_kernel()(nbr, x)


def _tc_body(x_ref, gs_ref, w1_ref, w2_ref, b_ref, out_ref, *, relu):
    h = (
        jnp.dot(x_ref[...], w1_ref[...], precision="highest",
                preferred_element_type=jnp.float32)
        + jnp.dot(gs_ref[...] * (1.0 / K), w2_ref[...], precision="highest",
                  preferred_element_type=jnp.float32)
        + b_ref[...]
    )
    out_ref[...] = jnp.maximum(h, 0.0) if relu else h


def _tc_body_resid(x_ref, gs_ref, w1_ref, w2_ref, b_ref, t_ref, out_ref):
    h = (
        jnp.dot(x_ref[...], w1_ref[...], precision="highest",
                preferred_element_type=jnp.float32)
        + jnp.dot(gs_ref[...] * (1.0 / K), w2_ref[...], precision="highest",
                  preferred_element_type=jnp.float32)
        + b_ref[...]
    )
    out_ref[...] = (t_ref[...] + jnp.maximum(h, 0.0)) * 0.5


_BLK = 1024


def _combine(x, gs, w1, w2, b, relu, temp=None):
    dout = w1.shape[1]
    row_spec = pl.BlockSpec((_BLK, D), lambda i: (i, 0))
    w_spec = pl.BlockSpec(w1.shape, lambda i: (0, 0))
    b_spec = pl.BlockSpec((1, dout), lambda i: (0, 0))
    out_spec = pl.BlockSpec((_BLK, dout), lambda i: (i, 0))
    if temp is None:
        return pl.pallas_call(
            functools.partial(_tc_body, relu=relu),
            grid=(N_PAD // _BLK,),
            in_specs=[row_spec, row_spec, w_spec, w_spec, b_spec],
            out_specs=out_spec,
            out_shape=jax.ShapeDtypeStruct((N_PAD, dout), jnp.float32),
        )(x, gs, w1, w2, b.reshape(1, dout))
    return pl.pallas_call(
        _tc_body_resid,
        grid=(N_PAD // _BLK,),
        in_specs=[row_spec, row_spec, w_spec, w_spec, b_spec, row_spec],
        out_specs=out_spec,
        out_shape=jax.ShapeDtypeStruct((N_PAD, dout), jnp.float32),
    )(x, gs, w1, w2, b.reshape(1, dout), temp)


def kernel(neighbours, shape_features, W1s, W2s, bs, W1_out, W2_out, b_out):
    nbr = jnp.asarray(neighbours, jnp.int32)
    nbr_pad = jnp.zeros((N_PAD, K), jnp.int32).at[:N].set(nbr)
    nbr_pad = nbr_pad.reshape(N_PAD * K // 128, 128)
    x = jnp.zeros((N_PAD, D), jnp.float32).at[:N].set(shape_features)

    def gcn(x, w1, w2, b, relu, temp=None):
        gs = _gather_sum(nbr_pad, x)
        return _combine(x, gs, w1, w2, b, relu, temp)

    x = gcn(x, W1s[0], W2s[0], bs[0], True)
    for i in range(1, 12, 2):
        t = x
        x = gcn(x, W1s[i], W2s[i], bs[i], True)
        x = gcn(x, W1s[i + 1], W2s[i + 1], bs[i + 1], True, temp=t)

    w1o = jnp.zeros((D, 128), jnp.float32).at[:, :3].set(W1_out)
    w2o = jnp.zeros((D, 128), jnp.float32).at[:, :3].set(W2_out)
    bo = jnp.zeros((128,), jnp.float32).at[:3].set(b_out)
    coords = gcn(x, w1o, w2o, bo, False)
    return (x[:N], coords[:N, :3])


# speedup vs baseline: 1.5312x; 1.0016x over previous
import functools

import jax
import jax.numpy as jnp
from jax import lax
from jax.experimental import pallas as pl
from jax.experimental.pallas import tpu as pltpu
from jax.experimental.pallas import tpu_sc as plsc

N, K, D = 10000, 32, 128
NC, NS, L = 2, 16, 16
NW = NC * NS
N_PAD = 10240
CPW = N_PAD // NW
ROWS_PER_STEP = 128
NODES_PER_STEP = ROWS_PER_STEP // K
STEPS = CPW * K // ROWS_PER_STEP
NBUF = 4
NJ = STEPS // NBUF


def _sc_body(nbr_ref, x_ref, out_ref, idx_v, rows_v, out_v, *sems):
    wid = lax.axis_index("s") * NC + lax.axis_index("c")
    base = wid * CPW

    pltpu.sync_copy(nbr_ref.at[pl.ds(wid * STEPS, STEPS)], idx_v)
    for b in range(NBUF - 1):
        pltpu.async_copy(x_ref.at[idx_v.at[b]], rows_v.at[b], sems[b])

    def compute_step(j, b):
        pltpu.make_async_copy(x_ref.at[idx_v.at[j]], rows_v.at[b], sems[b]).wait()
        rows = rows_v.at[b]

        def node_body(g, carry):
            node = j * NODES_PER_STEP + g
            r0 = g * K
            nd = D // L
            for d0 in range(0, nd, 4):
                ds_ = [pl.ds(d * L, L) for d in range(d0, d0 + 4)]
                accs = [rows[r0, s] for s in ds_]
                for k in range(1, K):
                    accs = [a + rows[r0 + k, s] for a, s in zip(accs, ds_)]
                for s, a in zip(ds_, accs):
                    out_v[node, s] = a
            return carry

        lax.fori_loop(0, NODES_PER_STEP, node_body, 0)

    def loop_body(jj, carry):
        for b in range(NBUF):
            j = jj * NBUF + b
            compute_step(j, b)
            nxt = j + NBUF - 1
            nb = (b + NBUF - 1) % NBUF

            @pl.when(nxt < STEPS)
            def _():
                pltpu.async_copy(x_ref.at[idx_v.at[nxt]], rows_v.at[nb],
                                 sems[nb])

        return carry

    lax.fori_loop(0, NJ, loop_body, 0)
    pltpu.sync_copy(out_v, out_ref.at[pl.ds(base, CPW)])


@functools.cache
def _gather_sum_kernel():
    return pl.kernel(
        _sc_body,
        out_type=jax.ShapeDtypeStruct((N_PAD, D), jnp.float32),
        mesh=plsc.VectorSubcoreMesh(core_axis_name="c", subcore_axis_name="s"),
        scratch_types=[
            pltpu.VMEM((STEPS, 128), jnp.int32),
            pltpu.VMEM((NBUF, ROWS_PER_STEP, D), jnp.float32),
            pltpu.VMEM((CPW, D), jnp.float32),
        ] + [pltpu.SemaphoreType.DMA] * NBUF,
    )


def _gather_sum(nbr, x):
    return _gather_sum_kernel()(nbr, x)


def _tc_body(x_ref, gs_ref, w1_ref, w2_ref, b_ref, out_ref, *, relu):
    h = (
        jnp.dot(x_ref[...], w1_ref[...], precision="highest",
                preferred_element_type=jnp.float32)
        + jnp.dot(gs_ref[...] * (1.0 / K), w2_ref[...], precision="highest",
                  preferred_element_type=jnp.float32)
        + b_ref[...]
    )
    out_ref[...] = jnp.maximum(h, 0.0) if relu else h


def _tc_body_resid(x_ref, gs_ref, w1_ref, w2_ref, b_ref, t_ref, out_ref):
    h = (
        jnp.dot(x_ref[...], w1_ref[...], precision="highest",
                preferred_element_type=jnp.float32)
        + jnp.dot(gs_ref[...] * (1.0 / K), w2_ref[...], precision="highest",
                  preferred_element_type=jnp.float32)
        + b_ref[...]
    )
    out_ref[...] = (t_ref[...] + jnp.maximum(h, 0.0)) * 0.5


_BLK = 1024


def _combine(x, gs, w1, w2, b, relu, temp=None):
    dout = w1.shape[1]
    row_spec = pl.BlockSpec((_BLK, D), lambda i: (i, 0))
    w_spec = pl.BlockSpec(w1.shape, lambda i: (0, 0))
    b_spec = pl.BlockSpec((1, dout), lambda i: (0, 0))
    out_spec = pl.BlockSpec((_BLK, dout), lambda i: (i, 0))
    if temp is None:
        return pl.pallas_call(
            functools.partial(_tc_body, relu=relu),
            grid=(N_PAD // _BLK,),
            in_specs=[row_spec, row_spec, w_spec, w_spec, b_spec],
            out_specs=out_spec,
            out_shape=jax.ShapeDtypeStruct((N_PAD, dout), jnp.float32),
        )(x, gs, w1, w2, b.reshape(1, dout))
    return pl.pallas_call(
        _tc_body_resid,
        grid=(N_PAD // _BLK,),
        in_specs=[row_spec, row_spec, w_spec, w_spec, b_spec, row_spec],
        out_specs=out_spec,
        out_shape=jax.ShapeDtypeStruct((N_PAD, dout), jnp.float32),
    )(x, gs, w1, w2, b.reshape(1, dout), temp)


def kernel(neighbours, shape_features, W1s, W2s, bs, W1_out, W2_out, b_out):
    nbr = jnp.asarray(neighbours, jnp.int32)
    nbr_pad = jnp.zeros((N_PAD, K), jnp.int32).at[:N].set(nbr)
    nbr_pad = nbr_pad.reshape(N_PAD * K // 128, 128)
    x = jnp.zeros((N_PAD, D), jnp.float32).at[:N].set(shape_features)

    def gcn(x, w1, w2, b, relu, temp=None):
        gs = _gather_sum(nbr_pad, x)
        return _combine(x, gs, w1, w2, b, relu, temp)

    x = gcn(x, W1s[0], W2s[0], bs[0], True)
    for i in range(1, 12, 2):
        t = x
        x = gcn(x, W1s[i], W2s[i], bs[i], True)
        x = gcn(x, W1s[i + 1], W2s[i + 1], bs[i + 1], True, temp=t)

    w1o = jnp.zeros((D, 128), jnp.float32).at[:, :3].set(W1_out)
    w2o = jnp.zeros((D, 128), jnp.float32).at[:, :3].set(W2_out)
    bo = jnp.zeros((128,), jnp.float32).at[:3].set(b_out)
    coords = gcn(x, w1o, w2o, bo, False)
    return (x[:N], coords[:N, :3])


# speedup vs baseline: 1.7995x; 1.1753x over previous
import functools

import jax
import jax.numpy as jnp
from jax import lax
from jax.experimental import pallas as pl
from jax.experimental.pallas import tpu as pltpu
from jax.experimental.pallas import tpu_sc as plsc

N, K, D = 10000, 32, 128
NC, NS, L = 2, 16, 16
NW = NC * NS
N_PAD = 10240
HALF = N_PAD // NC
PAIRS = HALF // 2
P = 128
NCH = PAIRS // P
NG = D // 8


def _sc_body(pidx_ref, xblk_ref, agg_ref, tbl_v, idx_v, out_v,
             si0, si1, so0, so1):
    c = lax.axis_index("c")
    s = lax.axis_index("s")
    pbase = c * PAIRS

    pltpu.sync_copy(xblk_ref.at[s], tbl_v)

    iota = lax.broadcasted_iota(jnp.int32, (L,), 0)
    halfv = jnp.where(iota >= 8, 1, 0)
    colv = iota & 7
    perms = [2 * m + halfv for m in range(8)]

    isems = (si0, si1)
    osems = (so0, so1)

    def idx_copy(ch, b):
        return pltpu.make_async_copy(
            pidx_ref.at[pl.ds(pbase + ch * P, P)], idx_v.at[b], isems[b])

    def out_copy(ch, b):
        dst = agg_ref.at[s, pl.ds(c * (HALF * 8) + ch * (P * 16), P * 16)]
        return pltpu.make_async_copy(out_v.at[b], dst, osems[b])

    idx_copy(0, 0).start()
    idx_copy(1, 1).start()

    def chunk(cc, ch, b):
        idx_copy(ch, b).wait()

        @pl.when(cc > 0)
        def _():
            out_copy(ch, b).wait()

        def pair_body(p, carry):
            acc = None
            for kg in range(4):
                vkg = idx_v[b, p, pl.ds(kg * 16, 16)]
                for m in range(8):
                    t = jnp.take_along_axis(vkg, perms[m], axis=0)
                    val = plsc.load_gather(tbl_v, [t + colv])
                    acc = val if acc is None else acc + val
            out_v[b, pl.ds(p * 16, 16)] = acc
            return carry

        lax.fori_loop(0, P, pair_body, 0)
        out_copy(ch, b).start()

        @pl.when(ch + 2 < NCH)
        def _():
            idx_copy(ch + 2, b).start()

    def loop_body(cc, carry):
        chunk(cc, 2 * cc, 0)
        chunk(cc, 2 * cc + 1, 1)
        return carry

    lax.fori_loop(0, NCH // 2, loop_body, 0)
    out_copy(NCH - 2, 0).wait()
    out_copy(NCH - 1, 1).wait()


@functools.cache
def _gather_sum_kernel():
    return pl.kernel(
        _sc_body,
        out_type=jax.ShapeDtypeStruct((NG, N_PAD * 8), jnp.float32),
        mesh=plsc.VectorSubcoreMesh(core_axis_name="c", subcore_axis_name="s"),
        compiler_params=pltpu.CompilerParams(needs_layout_passes=False),
        scratch_types=[
            pltpu.VMEM((N_PAD * 8,), jnp.float32),
            pltpu.VMEM((2, P, 2 * K), jnp.int32),
            pltpu.VMEM((2, P * 16), jnp.float32),
            pltpu.SemaphoreType.DMA,
            pltpu.SemaphoreType.DMA,
            pltpu.SemaphoreType.DMA,
            pltpu.SemaphoreType.DMA,
        ],
    )


def _gather_sum(pidx, xblk):
    return _gather_sum_kernel()(pidx, xblk)


def _dense(x, gsb, w1_ref, w2_ref, b_ref):
    w2b = w2_ref[...].reshape(NG, 8, D)
    part = lax.dot_general(
        gsb * (1.0 / K), w2b, (((2,), (1,)), ((0,), (0,))),
        precision="highest", preferred_element_type=jnp.float32)
    return (
        jnp.dot(x, w1_ref[...], precision="highest",
                preferred_element_type=jnp.float32)
        + jnp.sum(part, axis=0)
        + b_ref[...]
    )


def _blocked(out):
    return out.reshape(-1, NG, 8).transpose(1, 0, 2)


def _tc_body(x_ref, gs_ref, w1_ref, w2_ref, b_ref, out_ref, blk_ref, *, relu):
    h = _dense(x_ref[...], gs_ref[...], w1_ref, w2_ref, b_ref)
    out = jnp.maximum(h, 0.0) if relu else h
    out_ref[...] = out
    blk_ref[...] = _blocked(out)


def _tc_body_resid(x_ref, gs_ref, w1_ref, w2_ref, b_ref, t_ref, out_ref,
                   blk_ref):
    h = _dense(x_ref[...], gs_ref[...], w1_ref, w2_ref, b_ref)
    out = (t_ref[...] + jnp.maximum(h, 0.0)) * 0.5
    out_ref[...] = out
    blk_ref[...] = _blocked(out)


_BLK = 1024


def _combine(x, gs, w1, w2, b, relu, temp=None):
    gsb = gs.reshape(NG, N_PAD, 8)
    row_spec = pl.BlockSpec((_BLK, D), lambda i: (i, 0))
    gs_spec = pl.BlockSpec((NG, _BLK, 8), lambda i: (0, i, 0))
    w_spec = pl.BlockSpec((D, D), lambda i: (0, 0))
    b_spec = pl.BlockSpec((1, D), lambda i: (0, 0))
    out_shapes = (
        jax.ShapeDtypeStruct((N_PAD, D), jnp.float32),
        jax.ShapeDtypeStruct((NG, N_PAD, 8), jnp.float32),
    )
    out_specs = (row_spec, gs_spec)
    if temp is None:
        out, blk = pl.pallas_call(
            functools.partial(_tc_body, relu=relu),
            grid=(N_PAD // _BLK,),
            in_specs=[row_spec, gs_spec, w_spec, w_spec, b_spec],
            out_specs=out_specs,
            out_shape=out_shapes,
        )(x, gsb, w1, w2, b.reshape(1, D))
    else:
        out, blk = pl.pallas_call(
            _tc_body_resid,
            grid=(N_PAD // _BLK,),
            in_specs=[row_spec, gs_spec, w_spec, w_spec, b_spec, row_spec],
            out_specs=out_specs,
            out_shape=out_shapes,
        )(x, gsb, w1, w2, b.reshape(1, D), temp)
    return out, blk.reshape(NG, N_PAD * 8)


def kernel(neighbours, shape_features, W1s, W2s, bs, W1_out, W2_out, b_out):
    nbr = jnp.asarray(neighbours, jnp.int32)
    nbr_pad = jnp.zeros((N_PAD, K), jnp.int32).at[:N].set(nbr)
    pidx = (nbr_pad * 8).reshape(N_PAD // 2, 2, K).transpose(0, 2, 1)
    pidx = pidx.reshape(N_PAD // 2, 2 * K)

    x = jnp.zeros((N_PAD, D), jnp.float32).at[:N].set(shape_features)
    xblk = x.reshape(N_PAD, NG, 8).transpose(1, 0, 2).reshape(NG, N_PAD * 8)

    def gcn(x, xblk, w1, w2, b, relu, temp=None):
        gs = _gather_sum(pidx, xblk)
        return _combine(x, gs, w1, w2, b, relu, temp)

    x, xblk = gcn(x, xblk, W1s[0], W2s[0], bs[0], True)
    for i in range(1, 12, 2):
        t = x
        x, xblk = gcn(x, xblk, W1s[i], W2s[i], bs[i], True)
        x, xblk = gcn(x, xblk, W1s[i + 1], W2s[i + 1], bs[i + 1], True, temp=t)

    w1o = jnp.zeros((D, D), jnp.float32).at[:, :3].set(W1_out)
    w2o = jnp.zeros((D, D), jnp.float32).at[:, :3].set(W2_out)
    bo = jnp.zeros((D,), jnp.float32).at[:3].set(b_out)
    coords, _ = gcn(x, xblk, w1o, w2o, bo, False)
    return (x[:N], coords[:N, :3])


# speedup vs baseline: 2.6453x; 1.4700x over previous
import functools

import jax
import jax.numpy as jnp
from jax import lax
from jax.experimental import pallas as pl
from jax.experimental.pallas import tpu as pltpu
from jax.experimental.pallas import tpu_sc as plsc

N, K, D = 10000, 32, 128
NC, NS, L = 2, 16, 16
N_PAD = 10240
HALF = N_PAD // NC
PAIRS = HALF // 2
P = 128
CH_N = 2 * P
NCH = PAIRS // P
NR = D // NS


def _sc_body(pidx_ref, xt_ref, agg_ref, tbl_v, idx_v, out0_v, out1_v,
             si0, si1, so0, so1):
    outs = (out0_v, out1_v)
    c = lax.axis_index("c")
    s = lax.axis_index("s")
    pbase = c * PAIRS

    for r in range(NR):
        pltpu.sync_copy(xt_ref.at[s * NR + r],
                        tbl_v.at[pl.ds(r * N_PAD, N_PAD)])

    iota = lax.broadcasted_iota(jnp.int32, (L,), 0)
    halfv = jnp.where(iota >= 8, 1, 0)
    rowv = (iota & 7) * N_PAD
    colo = (iota & 7) * CH_N
    perms = [2 * m + halfv for m in range(8)]

    isems = (si0, si1)
    osems = (so0, so1)

    def idx_copy(ch, b):
        return pltpu.make_async_copy(
            pidx_ref.at[pl.ds(pbase + ch * P, P)], idx_v.at[b], isems[b])

    def out_copies(ch, b):
        col0 = c * HALF + ch * CH_N
        return [
            pltpu.make_async_copy(
                outs[b].at[pl.ds(r * CH_N, CH_N)],
                agg_ref.at[s * NR + r, pl.ds(col0, CH_N)],
                osems[b])
            for r in range(NR)
        ]

    idx_copy(0, 0).start()
    idx_copy(1, 1).start()

    def chunk(cc, ch, b):
        idx_copy(ch, b).wait()

        @pl.when(cc > 0)
        def _():
            for cp in out_copies(ch, b):
                cp.wait()

        def pair_body(p, carry):
            accs = [None] * 4
            for kg in range(4):
                vkg = idx_v[b, p, pl.ds(kg * 16, 16)]
                for m in range(8):
                    t = jnp.take_along_axis(vkg, perms[m], axis=0)
                    val = plsc.load_gather(tbl_v, [t + rowv])
                    accs[kg] = val if accs[kg] is None else accs[kg] + val
            acc = (accs[0] + accs[1]) + (accs[2] + accs[3])
            nloc = (2 * p + halfv) + colo
            plsc.store_scatter(outs[b], [nloc], acc)
            return carry

        lax.fori_loop(0, P, pair_body, 0)
        for cp in out_copies(ch, b):
            cp.start()

        @pl.when(ch + 2 < NCH)
        def _():
            idx_copy(ch + 2, b).start()

    def loop_body(cc, carry):
        chunk(cc, 2 * cc, 0)
        chunk(cc, 2 * cc + 1, 1)
        return carry

    lax.fori_loop(0, NCH // 2, loop_body, 0)
    for cp in out_copies(NCH - 2, 0):
        cp.wait()
    for cp in out_copies(NCH - 1, 1):
        cp.wait()


@functools.cache
def _gather_sum_kernel():
    return pl.kernel(
        _sc_body,
        out_type=jax.ShapeDtypeStruct((D, N_PAD), jnp.float32),
        mesh=plsc.VectorSubcoreMesh(core_axis_name="c", subcore_axis_name="s"),
        compiler_params=pltpu.CompilerParams(needs_layout_passes=False),
        scratch_types=[
            pltpu.VMEM((NR * N_PAD,), jnp.float32),
            pltpu.VMEM((2, P, 2 * K), jnp.int32),
            pltpu.VMEM((NR * CH_N,), jnp.float32),
            pltpu.VMEM((NR * CH_N,), jnp.float32),
            pltpu.SemaphoreType.DMA,
            pltpu.SemaphoreType.DMA,
            pltpu.SemaphoreType.DMA,
            pltpu.SemaphoreType.DMA,
        ],
    )


def _gather_sum(pidx, xt):
    return _gather_sum_kernel()(pidx, xt)


def _dense_t(xt_ref, gt_ref, w1_ref, w2_ref, b_ref):
    cd = (((0,), (0,)), ((), ()))
    return (
        lax.dot_general(w1_ref[...], xt_ref[...], cd, precision="highest",
                        preferred_element_type=jnp.float32)
        + lax.dot_general(w2_ref[...], gt_ref[...] * (1.0 / K), cd,
                          precision="highest",
                          preferred_element_type=jnp.float32)
        + b_ref[...]
    )


def _tc_body(xt_ref, gt_ref, w1_ref, w2_ref, b_ref, out_ref, *, relu):
    h = _dense_t(xt_ref, gt_ref, w1_ref, w2_ref, b_ref)
    out_ref[...] = jnp.maximum(h, 0.0) if relu else h


def _tc_body_resid(xt_ref, gt_ref, w1_ref, w2_ref, b_ref, t_ref, out_ref):
    h = _dense_t(xt_ref, gt_ref, w1_ref, w2_ref, b_ref)
    out_ref[...] = (t_ref[...] + jnp.maximum(h, 0.0)) * 0.5


_BLK = 1024


def _combine(xt, gt, w1, w2, b, relu, temp=None):
    col_spec = pl.BlockSpec((D, _BLK), lambda i: (0, i))
    w_spec = pl.BlockSpec((D, D), lambda i: (0, 0))
    b_spec = pl.BlockSpec((D, 1), lambda i: (0, 0))
    out_shape = jax.ShapeDtypeStruct((D, N_PAD), jnp.float32)
    if temp is None:
        return pl.pallas_call(
            functools.partial(_tc_body, relu=relu),
            grid=(N_PAD // _BLK,),
            in_specs=[col_spec, col_spec, w_spec, w_spec, b_spec],
            out_specs=col_spec,
            out_shape=out_shape,
        )(xt, gt, w1, w2, b.reshape(D, 1))
    return pl.pallas_call(
        _tc_body_resid,
        grid=(N_PAD // _BLK,),
        in_specs=[col_spec, col_spec, w_spec, w_spec, b_spec, col_spec],
        out_specs=col_spec,
        out_shape=out_shape,
    )(xt, gt, w1, w2, b.reshape(D, 1), temp)


def kernel(neighbours, shape_features, W1s, W2s, bs, W1_out, W2_out, b_out):
    nbr = jnp.asarray(neighbours, jnp.int32)
    nbr_pad = jnp.zeros((N_PAD, K), jnp.int32).at[:N].set(nbr)
    pidx = nbr_pad.reshape(N_PAD // 2, 2, K).transpose(0, 2, 1)
    pidx = pidx.reshape(N_PAD // 2, 2 * K)

    xt = jnp.zeros((D, N_PAD), jnp.float32).at[:, :N].set(shape_features.T)

    def gcn(xt, w1, w2, b, relu, temp=None):
        gt = _gather_sum(pidx, xt)
        return _combine(xt, gt, w1, w2, b, relu, temp)

    xt = gcn(xt, W1s[0], W2s[0], bs[0], True)
    for i in range(1, 12, 2):
        t = xt
        xt = gcn(xt, W1s[i], W2s[i], bs[i], True)
        xt = gcn(xt, W1s[i + 1], W2s[i + 1], bs[i + 1], True, temp=t)

    w1o = jnp.zeros((D, D), jnp.float32).at[:, :3].set(W1_out)
    w2o = jnp.zeros((D, D), jnp.float32).at[:, :3].set(W2_out)
    bo = jnp.zeros((D,), jnp.float32).at[:3].set(b_out)
    coords_t = gcn(xt, w1o, w2o, bo, False)
    return (xt.T[:N], coords_t.T[:N, :3])


# speedup vs baseline: 2.7785x; 1.0504x over previous
import functools

import jax
import jax.numpy as jnp
from jax import lax
from jax.experimental import pallas as pl
from jax.experimental.pallas import tpu as pltpu
from jax.experimental.pallas import tpu_sc as plsc

N, K, D = 10000, 32, 128
NC, NS, L = 2, 16, 16
N_PAD = 10240
HALF = N_PAD // NC
PAIRS = HALF // 2
P = 128
CH_N = 2 * P
NCH = PAIRS // P
NR = D // NS


def _sc_body(pidx_ref, xt_ref, agg_ref, tbl_v, idx_v, out0_v, out1_v,
             si0, si1, so0, so1):
    outs = (out0_v, out1_v)
    c = lax.axis_index("c")
    s = lax.axis_index("s")
    pbase = c * PAIRS

    for r in range(NR):
        pltpu.sync_copy(xt_ref.at[s * NR + r],
                        tbl_v.at[pl.ds(r * N_PAD, N_PAD)])

    iota = lax.broadcasted_iota(jnp.int32, (L,), 0)
    halfv = jnp.where(iota >= 8, 1, 0)
    rowv = (iota & 7) * N_PAD
    colo = (iota & 7) * CH_N
    perms = [2 * m + halfv for m in range(8)]

    isems = (si0, si1)
    osems = (so0, so1)

    def idx_copy(ch, b):
        return pltpu.make_async_copy(
            pidx_ref.at[pl.ds(pbase + ch * P, P)], idx_v.at[b], isems[b])

    def out_copies(ch, b):
        col0 = c * HALF + ch * CH_N
        return [
            pltpu.make_async_copy(
                outs[b].at[pl.ds(r * CH_N, CH_N)],
                agg_ref.at[s * NR + r, pl.ds(col0, CH_N)],
                osems[b])
            for r in range(NR)
        ]

    idx_copy(0, 0).start()
    idx_copy(1, 1).start()

    def chunk(cc, ch, b):
        idx_copy(ch, b).wait()

        @pl.when(cc > 0)
        def _():
            for cp in out_copies(ch, b):
                cp.wait()

        @plsc.parallel_loop(0, P, step=1, unroll=4)
        def pair_body(p):
            accs = [None] * 4
            for kg in range(4):
                vkg = idx_v[b, p, pl.ds(kg * 16, 16)]
                for m in range(8):
                    t = jnp.take_along_axis(vkg, perms[m], axis=0)
                    val = plsc.load_gather(tbl_v, [t + rowv])
                    accs[kg] = val if accs[kg] is None else accs[kg] + val
            acc = (accs[0] + accs[1]) + (accs[2] + accs[3])
            nloc = (2 * p + halfv) + colo
            plsc.store_scatter(outs[b], [nloc], acc)
        for cp in out_copies(ch, b):
            cp.start()

        @pl.when(ch + 2 < NCH)
        def _():
            idx_copy(ch + 2, b).start()

    def loop_body(cc, carry):
        chunk(cc, 2 * cc, 0)
        chunk(cc, 2 * cc + 1, 1)
        return carry

    lax.fori_loop(0, NCH // 2, loop_body, 0)
    for cp in out_copies(NCH - 2, 0):
        cp.wait()
    for cp in out_copies(NCH - 1, 1):
        cp.wait()


@functools.cache
def _gather_sum_kernel():
    return pl.kernel(
        _sc_body,
        out_type=jax.ShapeDtypeStruct((D, N_PAD), jnp.float32),
        mesh=plsc.VectorSubcoreMesh(core_axis_name="c", subcore_axis_name="s"),
        compiler_params=pltpu.CompilerParams(needs_layout_passes=False),
        scratch_types=[
            pltpu.VMEM((NR * N_PAD,), jnp.float32),
            pltpu.VMEM((2, P, 2 * K), jnp.int32),
            pltpu.VMEM((NR * CH_N,), jnp.float32),
            pltpu.VMEM((NR * CH_N,), jnp.float32),
            pltpu.SemaphoreType.DMA,
            pltpu.SemaphoreType.DMA,
            pltpu.SemaphoreType.DMA,
            pltpu.SemaphoreType.DMA,
        ],
    )


def _gather_sum(pidx, xt):
    return _gather_sum_kernel()(pidx, xt)


def _dense_t(xt_ref, gt_ref, w1_ref, w2_ref, b_ref):
    cd = (((0,), (0,)), ((), ()))
    return (
        lax.dot_general(w1_ref[...], xt_ref[...], cd, precision="highest",
                        preferred_element_type=jnp.float32)
        + lax.dot_general(w2_ref[...], gt_ref[...] * (1.0 / K), cd,
                          precision="highest",
                          preferred_element_type=jnp.float32)
        + b_ref[...]
    )


def _tc_body(xt_ref, gt_ref, w1_ref, w2_ref, b_ref, out_ref, *, relu):
    h = _dense_t(xt_ref, gt_ref, w1_ref, w2_ref, b_ref)
    out_ref[...] = jnp.maximum(h, 0.0) if relu else h


def _tc_body_resid(xt_ref, gt_ref, w1_ref, w2_ref, b_ref, t_ref, out_ref):
    h = _dense_t(xt_ref, gt_ref, w1_ref, w2_ref, b_ref)
    out_ref[...] = (t_ref[...] + jnp.maximum(h, 0.0)) * 0.5


_BLK = 1024


def _combine(xt, gt, w1, w2, b, relu, temp=None):
    col_spec = pl.BlockSpec((D, _BLK), lambda i: (0, i))
    w_spec = pl.BlockSpec((D, D), lambda i: (0, 0))
    b_spec = pl.BlockSpec((D, 1), lambda i: (0, 0))
    out_shape = jax.ShapeDtypeStruct((D, N_PAD), jnp.float32)
    if temp is None:
        return pl.pallas_call(
            functools.partial(_tc_body, relu=relu),
            grid=(N_PAD // _BLK,),
            in_specs=[col_spec, col_spec, w_spec, w_spec, b_spec],
            out_specs=col_spec,
            out_shape=out_shape,
        )(xt, gt, w1, w2, b.reshape(D, 1))
    return pl.pallas_call(
        _tc_body_resid,
        grid=(N_PAD // _BLK,),
        in_specs=[col_spec, col_spec, w_spec, w_spec, b_spec, col_spec],
        out_specs=col_spec,
        out_shape=out_shape,
    )(xt, gt, w1, w2, b.reshape(D, 1), temp)


def kernel(neighbours, shape_features, W1s, W2s, bs, W1_out, W2_out, b_out):
    nbr = jnp.asarray(neighbours, jnp.int32)
    nbr_pad = jnp.zeros((N_PAD, K), jnp.int32).at[:N].set(nbr)
    pidx = nbr_pad.reshape(N_PAD // 2, 2, K).transpose(0, 2, 1)
    pidx = pidx.reshape(N_PAD // 2, 2 * K)

    xt = jnp.zeros((D, N_PAD), jnp.float32).at[:, :N].set(shape_features.T)

    def gcn(xt, w1, w2, b, relu, temp=None):
        gt = _gather_sum(pidx, xt)
        return _combine(xt, gt, w1, w2, b, relu, temp)

    xt = gcn(xt, W1s[0], W2s[0], bs[0], True)
    for i in range(1, 12, 2):
        t = xt
        xt = gcn(xt, W1s[i], W2s[i], bs[i], True)
        xt = gcn(xt, W1s[i + 1], W2s[i + 1], bs[i + 1], True, temp=t)

    w1o = jnp.zeros((D, D), jnp.float32).at[:, :3].set(W1_out)
    w2o = jnp.zeros((D, D), jnp.float32).at[:, :3].set(W2_out)
    bo = jnp.zeros((D,), jnp.float32).at[:3].set(b_out)
    coords_t = gcn(xt, w1o, w2o, bo, False)
    return (xt.T[:N], coords_t.T[:N, :3])


# speedup vs baseline: 3.9379x; 1.4173x over previous
import functools

import jax
import jax.numpy as jnp
from jax import lax
from jax.experimental import pallas as pl
from jax.experimental.pallas import tpu as pltpu
from jax.experimental.pallas import tpu_sc as plsc

N, K, D = 10000, 32, 128
NC, NS, L = 2, 16, 16
N_PAD = 10240
HALF = N_PAD // NC
PAIRS = HALF // 2
P = 128
CH_N = 2 * P
NCH = PAIRS // P
NR = D // NS


def _sc_body(pidx_ref, xb_ref, agg_ref, tbl_v, idx_v, out0_v, out1_v,
             si0, si1, so0, so1):
    outs = (out0_v, out1_v)
    c = lax.axis_index("c")
    s = lax.axis_index("s")
    pbase = c * PAIRS

    pltpu.sync_copy(xb_ref.at[s], tbl_v)

    iota = lax.broadcasted_iota(jnp.int32, (L,), 0)
    rowv = iota & 7
    perms = [2 * m + jnp.where(iota >= 8, 1, 0) for m in range(8)]

    isems = (si0, si1)
    osems = (so0, so1)

    def idx_copy(ch, b):
        return pltpu.make_async_copy(
            pidx_ref.at[pl.ds(pbase + ch * P, P)], idx_v.at[b], isems[b])

    def out_copy(ch, b):
        node0 = c * HALF + ch * CH_N
        return pltpu.make_async_copy(
            outs[b], agg_ref.at[s, pl.ds(node0 * NR, CH_N * NR)], osems[b])

    idx_copy(0, 0).start()
    idx_copy(1, 1).start()

    def chunk(cc, ch, b):
        idx_copy(ch, b).wait()

        @pl.when(cc > 0)
        def _():
            out_copy(ch, b).wait()

        @plsc.parallel_loop(0, P, step=1, unroll=4)
        def pair_body(p):
            accs = [None] * 4
            for kg in range(4):
                vkg = idx_v[b, p, pl.ds(kg * 16, 16)]
                for m in range(8):
                    t = jnp.take_along_axis(vkg, perms[m], axis=0)
                    val = plsc.load_gather(tbl_v, [t + rowv])
                    accs[kg] = val if accs[kg] is None else accs[kg] + val
            acc = (accs[0] + accs[1]) + (accs[2] + accs[3])
            outs[b][pl.ds(p * 16, 16)] = acc

        out_copy(ch, b).start()

        @pl.when(ch + 2 < NCH)
        def _():
            idx_copy(ch + 2, b).start()

    def loop_body(cc, carry):
        chunk(cc, 2 * cc, 0)
        chunk(cc, 2 * cc + 1, 1)
        return carry

    lax.fori_loop(0, NCH // 2, loop_body, 0)
    out_copy(NCH - 2, 0).wait()
    out_copy(NCH - 1, 1).wait()


@functools.cache
def _gather_sum_kernel():
    return pl.kernel(
        _sc_body,
        out_type=jax.ShapeDtypeStruct((NS, N_PAD * NR), jnp.float32),
        mesh=plsc.VectorSubcoreMesh(core_axis_name="c", subcore_axis_name="s"),
        compiler_params=pltpu.CompilerParams(needs_layout_passes=False),
        scratch_types=[
            pltpu.VMEM((N_PAD * NR,), jnp.float32),
            pltpu.VMEM((2, P, 2 * K), jnp.int32),
            pltpu.VMEM((CH_N * NR,), jnp.float32),
            pltpu.VMEM((CH_N * NR,), jnp.float32),
            pltpu.SemaphoreType.DMA,
            pltpu.SemaphoreType.DMA,
            pltpu.SemaphoreType.DMA,
            pltpu.SemaphoreType.DMA,
        ],
    )


def _gather_sum(pidx, xb):
    return _gather_sum_kernel()(pidx, xb)


def _to_blocked(xt):
    return xt.reshape(NS, NR, N_PAD).transpose(0, 2, 1).reshape(
        NS, N_PAD * NR)


def _to_transposed(xb):
    return xb.reshape(NS, N_PAD, NR).transpose(0, 2, 1).reshape(D, N_PAD)


def _dense_t(xt_ref, gt_ref, w1_ref, w2_ref, b_ref):
    cd = (((0,), (0,)), ((), ()))
    return (
        lax.dot_general(w1_ref[...], xt_ref[...], cd, precision="highest",
                        preferred_element_type=jnp.float32)
        + lax.dot_general(w2_ref[...], gt_ref[...] * (1.0 / K), cd,
                          precision="highest",
                          preferred_element_type=jnp.float32)
        + b_ref[...]
    )


def _tc_body(xt_ref, gt_ref, w1_ref, w2_ref, b_ref, out_ref, *, relu):
    h = _dense_t(xt_ref, gt_ref, w1_ref, w2_ref, b_ref)
    out_ref[...] = jnp.maximum(h, 0.0) if relu else h


def _tc_body_resid(xt_ref, gt_ref, w1_ref, w2_ref, b_ref, t_ref, out_ref):
    h = _dense_t(xt_ref, gt_ref, w1_ref, w2_ref, b_ref)
    out_ref[...] = (t_ref[...] + jnp.maximum(h, 0.0)) * 0.5


_BLK = 1024


def _combine(xt, gt, w1, w2, b, relu, temp=None):
    col_spec = pl.BlockSpec((D, _BLK), lambda i: (0, i))
    w_spec = pl.BlockSpec((D, D), lambda i: (0, 0))
    b_spec = pl.BlockSpec((D, 1), lambda i: (0, 0))
    out_shape = jax.ShapeDtypeStruct((D, N_PAD), jnp.float32)
    if temp is None:
        return pl.pallas_call(
            functools.partial(_tc_body, relu=relu),
            grid=(N_PAD // _BLK,),
            in_specs=[col_spec, col_spec, w_spec, w_spec, b_spec],
            out_specs=col_spec,
            out_shape=out_shape,
        )(xt, gt, w1, w2, b.reshape(D, 1))
    return pl.pallas_call(
        _tc_body_resid,
        grid=(N_PAD // _BLK,),
        in_specs=[col_spec, col_spec, w_spec, w_spec, b_spec, col_spec],
        out_specs=col_spec,
        out_shape=out_shape,
    )(xt, gt, w1, w2, b.reshape(D, 1), temp)


def kernel(neighbours, shape_features, W1s, W2s, bs, W1_out, W2_out, b_out):
    nbr = jnp.asarray(neighbours, jnp.int32)
    nbr_pad = jnp.zeros((N_PAD, K), jnp.int32).at[:N].set(nbr)
    pidx = (nbr_pad * NR).reshape(N_PAD // 2, 2, K).transpose(0, 2, 1)
    pidx = pidx.reshape(N_PAD // 2, 2 * K)

    xt = jnp.zeros((D, N_PAD), jnp.float32).at[:, :N].set(shape_features.T)

    def gcn(xt, w1, w2, b, relu, temp=None):
        gt = _to_transposed(_gather_sum(pidx, _to_blocked(xt)))
        return _combine(xt, gt, w1, w2, b, relu, temp)

    xt = gcn(xt, W1s[0], W2s[0], bs[0], True)
    for i in range(1, 12, 2):
        t = xt
        xt = gcn(xt, W1s[i], W2s[i], bs[i], True)
        xt = gcn(xt, W1s[i + 1], W2s[i + 1], bs[i + 1], True, temp=t)

    w1o = jnp.zeros((D, D), jnp.float32).at[:, :3].set(W1_out)
    w2o = jnp.zeros((D, D), jnp.float32).at[:, :3].set(W2_out)
    bo = jnp.zeros((D,), jnp.float32).at[:3].set(b_out)
    coords_t = gcn(xt, w1o, w2o, bo, False)
    return (xt.T[:N], coords_t.T[:N, :3])
